# Initial kernel scaffold; baseline (speedup 1.0000x reference)
import jax
import jax.numpy as jnp
from jax.experimental import pallas as pl


def kernel(x, coords, w_q, w_k, w_v, w_rpe_w, w_rpe_b, w_o_w, w_o_b, ln1_s, ln1_b, ln2_s, ln2_b, ff_w1, ff_b1, ff_w2, ff_b2, alpha, beta):
    raise NotImplementedError("write your pallas kernel here")


# speedup vs baseline: 6.0256x; 6.0256x over previous
import functools

import jax
import jax.numpy as jnp
from jax import lax
from jax.experimental import pallas as pl

N = 16384
H = 8
D = 64
R = 2
BLK = 128
CDIM = 6
NW = 8
RH = R * H

TN = 1024
BB = 16


def _prep_body(x_ref, c_ref, wqt_ref, wkt_ref, wvt_ref, wpe_ref, bpe_ref,
               s_ref, b_ref, q_ref, k_ref, v_ref):
    x = x_ref[...]
    c = c_ref[...]
    m = jnp.mean(x, axis=-1, keepdims=True)
    v_ = jnp.mean((x - m) * (x - m), axis=-1, keepdims=True)
    xn = (x - m) / jnp.sqrt(v_ + 1e-5) * s_ref[...] + b_ref[...]

    pe = jnp.dot(c[:, :CDIM - 1], wpe_ref[...],
                 preferred_element_type=jnp.float32) + bpe_ref[...]
    qf = jnp.dot(xn, wqt_ref[...], preferred_element_type=jnp.float32) + pe
    kf = jnp.dot(xn, wkt_ref[...], preferred_element_type=jnp.float32) + pe
    vf = jnp.dot(xn, wvt_ref[...], preferred_element_type=jnp.float32)

    for h in range(H):
        q_ref[h] = qf[:, h * D:(h + 1) * D]
        k_ref[h] = kf[:, h * D:(h + 1) * D]
        v_ref[h] = vf[:, h * D:(h + 1) * D]


def _attn_body(q_ref, k_ref, v_ref, o_ref):
    for b in range(BB):
        q = q_ref[b]
        k = k_ref[b]
        s = lax.dot_general(q, k, (((1,), (1,)), ((), ())),
                            preferred_element_type=jnp.float32) * (1.0 / 8.0)
        mx = jnp.max(s, axis=-1, keepdims=True)
        p = jnp.exp(s - mx)
        l = jnp.sum(p, axis=-1, keepdims=True)
        o_ref[b] = jnp.dot(p / l, v_ref[b], preferred_element_type=jnp.float32)


def _epi_body(x_ref, o0_ref, o1_ref, wot_ref, wob_ref, s2_ref, b2_ref,
              w1t_ref, b1_ref, w2t_ref, b2f_ref, out_ref):
    acc = jnp.zeros((TN, D), jnp.float32)
    for h in range(H):
        oh = (o0_ref[h] + o1_ref[h]) * 0.5
        acc = acc + jnp.dot(oh, wot_ref[h], preferred_element_type=jnp.float32)
    x1 = x_ref[...] + acc + wob_ref[...]
    m = jnp.mean(x1, axis=-1, keepdims=True)
    v_ = jnp.mean((x1 - m) * (x1 - m), axis=-1, keepdims=True)
    x2 = (x1 - m) / jnp.sqrt(v_ + 1e-5) * s2_ref[...] + b2_ref[...]
    h1 = jnp.maximum(jnp.dot(x2, w1t_ref[...],
                             preferred_element_type=jnp.float32) + b1_ref[...], 0.0)
    ff = jnp.dot(h1, w2t_ref[...], preferred_element_type=jnp.float32) + b2f_ref[...]
    out_ref[...] = x1 + ff


def kernel(x, coords, w_q, w_k, w_v, w_rpe_w, w_rpe_b, w_o_w, w_o_b,
           ln1_s, ln1_b, ln2_s, ln2_b, ff_w1, ff_b1, ff_w2, ff_b2, alpha, beta):
    wqt = w_q.T
    wkt = w_k.T
    wvt = w_v.T
    wpe = w_rpe_w.reshape(H * D, CDIM - 1, NW).sum(-1).T
    bpe = w_rpe_b.reshape(1, H * D)
    wot = w_o_w.T.reshape(H, D, D)

    grid1 = N // TN
    q, k, v = pl.pallas_call(
        _prep_body,
        grid=(grid1,),
        in_specs=[
            pl.BlockSpec((TN, D), lambda i: (i, 0)),
            pl.BlockSpec((TN, CDIM), lambda i: (i, 0)),
            pl.BlockSpec((D, H * D), lambda i: (0, 0)),
            pl.BlockSpec((D, H * D), lambda i: (0, 0)),
            pl.BlockSpec((D, H * D), lambda i: (0, 0)),
            pl.BlockSpec((CDIM - 1, H * D), lambda i: (0, 0)),
            pl.BlockSpec((1, H * D), lambda i: (0, 0)),
            pl.BlockSpec((1, D), lambda i: (0, 0)),
            pl.BlockSpec((1, D), lambda i: (0, 0)),
        ],
        out_specs=[
            pl.BlockSpec((H, TN, D), lambda i: (0, i, 0)),
            pl.BlockSpec((H, TN, D), lambda i: (0, i, 0)),
            pl.BlockSpec((H, TN, D), lambda i: (0, i, 0)),
        ],
        out_shape=[
            jax.ShapeDtypeStruct((H, N, D), jnp.float32),
            jax.ShapeDtypeStruct((H, N, D), jnp.float32),
            jax.ShapeDtypeStruct((H, N, D), jnp.float32),
        ],
    )(x, coords, wqt, wkt, wvt, wpe, bpe,
      ln1_s.reshape(1, D), ln1_b.reshape(1, D))

    m_ = jnp.mean(x, axis=-1, keepdims=True)
    v_ = jnp.var(x, axis=-1, keepdims=True)
    xn_ = (x - m_) / jnp.sqrt(v_ + 1e-5) * ln1_s + ln1_b
    q_ = (xn_ @ w_q.T).reshape(N, H, D).transpose(1, 0, 2)
    dist_feat = jnp.repeat(coords[:, : CDIM - 1], NW, axis=1)
    pe_ = (dist_feat @ w_rpe_w.T + w_rpe_b).reshape(N, H, D).transpose(1, 0, 2)
    q_ = q_ + pe_
    qh = jnp.concatenate(
        [q_, jnp.broadcast_to(coords[None], (H, N, CDIM))], axis=-1)
    proj = jnp.einsum('hnd,rhd->rhn', qh, alpha) + beta[..., None]
    idx = jnp.argsort(proj, axis=-1)
    inv = jnp.argsort(idx, axis=-1)

    def gather(t):
        tb = jnp.broadcast_to(t[None], (R,) + t.shape)
        return jnp.take_along_axis(tb, idx[..., None], axis=2)

    qs = gather(q).reshape(RH * (N // BLK), BLK, D)
    ks = gather(k).reshape(RH * (N // BLK), BLK, D)
    vs = gather(v).reshape(RH * (N // BLK), BLK, D)

    nblocks = RH * (N // BLK)
    ob = pl.pallas_call(
        _attn_body,
        grid=(nblocks // BB,),
        in_specs=[pl.BlockSpec((BB, BLK, D), lambda i: (i, 0, 0))] * 3,
        out_specs=pl.BlockSpec((BB, BLK, D), lambda i: (i, 0, 0)),
        out_shape=jax.ShapeDtypeStruct((nblocks, BLK, D), jnp.float32),
    )(qs, ks, vs)

    ob = ob.reshape(R, H, N, D)
    obi = jnp.take_along_axis(ob, inv[..., None], axis=2)

    out = pl.pallas_call(
        _epi_body,
        grid=(grid1,),
        in_specs=[
            pl.BlockSpec((TN, D), lambda i: (i, 0)),
            pl.BlockSpec((H, TN, D), lambda i: (0, i, 0)),
            pl.BlockSpec((H, TN, D), lambda i: (0, i, 0)),
            pl.BlockSpec((H, D, D), lambda i: (0, 0, 0)),
            pl.BlockSpec((1, D), lambda i: (0, 0)),
            pl.BlockSpec((1, D), lambda i: (0, 0)),
            pl.BlockSpec((1, D), lambda i: (0, 0)),
            pl.BlockSpec((D, D), lambda i: (0, 0)),
            pl.BlockSpec((1, D), lambda i: (0, 0)),
            pl.BlockSpec((D, D), lambda i: (0, 0)),
            pl.BlockSpec((1, D), lambda i: (0, 0)),
        ],
        out_specs=pl.BlockSpec((TN, D), lambda i: (i, 0)),
        out_shape=jax.ShapeDtypeStruct((N, D), jnp.float32),
    )(x, obi[0], obi[1], wot, w_o_b.reshape(1, D),
      ln2_s.reshape(1, D), ln2_b.reshape(1, D),
      ff_w1.T, ff_b1.reshape(1, D), ff_w2.T, ff_b2.reshape(1, D))
    return out


# speedup vs baseline: 8.8813x; 1.4739x over previous
import functools

import jax
import jax.numpy as jnp
from jax import lax
from jax.experimental import pallas as pl
from jax.experimental.pallas import tpu as pltpu
from jax.experimental.pallas import tpu_sc as plsc

N = 16384
H = 8
D = 64
R = 2
BLK = 128
CDIM = 6
NW = 8
RH = R * H

TN = 1024
BB = 16

NWORK = 32
ROWS_PER_W = RH * N // NWORK
GW = 512
NWIN = ROWS_PER_W // GW

_SC_MESH = plsc.VectorSubcoreMesh(core_axis_name="c", subcore_axis_name="s")


@functools.partial(
    pl.kernel,
    mesh=_SC_MESH,
    out_type=[jax.ShapeDtypeStruct((RH * N, 2 * D), jnp.float32)] * 2,
    scratch_types=[
        pltpu.VMEM((GW,), jnp.int32),
        pltpu.VMEM((GW, 2 * D), jnp.float32),
        pltpu.SemaphoreType.DMA,
    ],
)
def _sc_gather2(qk_hbm, vv_hbm, idx_hbm, qks_hbm, vvs_hbm, idxv, buf, sem):
    wid = lax.axis_index("s") * 2 + lax.axis_index("c")
    base = wid * ROWS_PER_W
    for w in range(NWIN):
        row = base + w * GW
        pltpu.sync_copy(idx_hbm.at[pl.ds(row, GW)], idxv)
        for tab, dst in ((qk_hbm, qks_hbm), (vv_hbm, vvs_hbm)):
            pltpu.async_copy(tab.at[idxv], buf, sem).wait()
            pltpu.sync_copy(buf, dst.at[pl.ds(row, GW)])


@functools.partial(
    pl.kernel,
    mesh=_SC_MESH,
    out_type=jax.ShapeDtypeStruct((RH * N, 2 * D), jnp.float32),
    scratch_types=[
        pltpu.VMEM((GW,), jnp.int32),
        pltpu.VMEM((GW, 2 * D), jnp.float32),
        pltpu.SemaphoreType.DMA,
    ],
)
def _sc_scatter(ob_hbm, idx_hbm, out_hbm, idxv, buf, sem):
    wid = lax.axis_index("s") * 2 + lax.axis_index("c")
    base = wid * ROWS_PER_W
    for w in range(NWIN):
        row = base + w * GW
        pltpu.sync_copy(idx_hbm.at[pl.ds(row, GW)], idxv)
        pltpu.sync_copy(ob_hbm.at[pl.ds(row, GW)], buf)
        pltpu.async_copy(buf, out_hbm.at[idxv], sem).wait()


def _prep_body(x_ref, c_ref, wqt_ref, wkt_ref, wvt_ref, wpe_ref, bpe_ref,
               s_ref, b_ref, qk_ref, vv_ref):
    x = x_ref[...]
    c = c_ref[...]
    m = jnp.mean(x, axis=-1, keepdims=True)
    v_ = jnp.mean((x - m) * (x - m), axis=-1, keepdims=True)
    xn = (x - m) / jnp.sqrt(v_ + 1e-5) * s_ref[...] + b_ref[...]

    pe = jnp.dot(c[:, :CDIM - 1], wpe_ref[...],
                 preferred_element_type=jnp.float32) + bpe_ref[...]
    qf = jnp.dot(xn, wqt_ref[...], preferred_element_type=jnp.float32) + pe
    kf = jnp.dot(xn, wkt_ref[...], preferred_element_type=jnp.float32) + pe
    vf = jnp.dot(xn, wvt_ref[...], preferred_element_type=jnp.float32)

    for h in range(H):
        qk_ref[h, :, :D] = qf[:, h * D:(h + 1) * D]
        qk_ref[h, :, D:] = kf[:, h * D:(h + 1) * D]
        vv_ref[h, :, :D] = vf[:, h * D:(h + 1) * D]
        vv_ref[h, :, D:] = vf[:, h * D:(h + 1) * D]


def _attn_body(qk_ref, v_ref, o_ref):
    for b in range(BB):
        q = qk_ref[b, :, :D]
        k = qk_ref[b, :, D:]
        s = lax.dot_general(q, k, (((1,), (1,)), ((), ())),
                            preferred_element_type=jnp.float32) * (1.0 / 8.0)
        mx = jnp.max(s, axis=-1, keepdims=True)
        p = jnp.exp(s - mx)
        l = jnp.sum(p, axis=-1, keepdims=True)
        o = jnp.dot(p / l, v_ref[b, :, :D], preferred_element_type=jnp.float32)
        o_ref[b, :, :D] = o
        o_ref[b, :, D:] = o


def _epi_body(x_ref, o0_ref, o1_ref, wot_ref, wob_ref, s2_ref, b2_ref,
              w1t_ref, b1_ref, w2t_ref, b2f_ref, out_ref):
    acc = jnp.zeros((TN, D), jnp.float32)
    for h in range(H):
        oh = (o0_ref[h, :, :D] + o1_ref[h, :, :D]) * 0.5
        acc = acc + jnp.dot(oh, wot_ref[h], preferred_element_type=jnp.float32)
    x1 = x_ref[...] + acc + wob_ref[...]
    m = jnp.mean(x1, axis=-1, keepdims=True)
    v_ = jnp.mean((x1 - m) * (x1 - m), axis=-1, keepdims=True)
    x2 = (x1 - m) / jnp.sqrt(v_ + 1e-5) * s2_ref[...] + b2_ref[...]
    h1 = jnp.maximum(jnp.dot(x2, w1t_ref[...],
                             preferred_element_type=jnp.float32) + b1_ref[...], 0.0)
    ff = jnp.dot(h1, w2t_ref[...], preferred_element_type=jnp.float32) + b2f_ref[...]
    out_ref[...] = x1 + ff


def kernel(x, coords, w_q, w_k, w_v, w_rpe_w, w_rpe_b, w_o_w, w_o_b,
           ln1_s, ln1_b, ln2_s, ln2_b, ff_w1, ff_b1, ff_w2, ff_b2, alpha, beta):
    wqt = w_q.T
    wkt = w_k.T
    wvt = w_v.T
    wpe = w_rpe_w.reshape(H * D, CDIM - 1, NW).sum(-1).T
    bpe = w_rpe_b.reshape(1, H * D)
    wot = w_o_w.T.reshape(H, D, D)

    grid1 = N // TN
    qk, vv = pl.pallas_call(
        _prep_body,
        grid=(grid1,),
        in_specs=[
            pl.BlockSpec((TN, D), lambda i: (i, 0)),
            pl.BlockSpec((TN, CDIM), lambda i: (i, 0)),
            pl.BlockSpec((D, H * D), lambda i: (0, 0)),
            pl.BlockSpec((D, H * D), lambda i: (0, 0)),
            pl.BlockSpec((D, H * D), lambda i: (0, 0)),
            pl.BlockSpec((CDIM - 1, H * D), lambda i: (0, 0)),
            pl.BlockSpec((1, H * D), lambda i: (0, 0)),
            pl.BlockSpec((1, D), lambda i: (0, 0)),
            pl.BlockSpec((1, D), lambda i: (0, 0)),
        ],
        out_specs=[
            pl.BlockSpec((H, TN, 2 * D), lambda i: (0, i, 0)),
            pl.BlockSpec((H, TN, 2 * D), lambda i: (0, i, 0)),
        ],
        out_shape=[
            jax.ShapeDtypeStruct((H, N, 2 * D), jnp.float32),
            jax.ShapeDtypeStruct((H, N, 2 * D), jnp.float32),
        ],
    )(x, coords, wqt, wkt, wvt, wpe, bpe,
      ln1_s.reshape(1, D), ln1_b.reshape(1, D))

    m_ = jnp.mean(x, axis=-1, keepdims=True)
    v_ = jnp.var(x, axis=-1, keepdims=True)
    xn_ = (x - m_) / jnp.sqrt(v_ + 1e-5) * ln1_s + ln1_b
    q_ = (xn_ @ w_q.T).reshape(N, H, D).transpose(1, 0, 2)
    dist_feat = jnp.repeat(coords[:, : CDIM - 1], NW, axis=1)
    pe_ = (dist_feat @ w_rpe_w.T + w_rpe_b).reshape(N, H, D).transpose(1, 0, 2)
    q_ = q_ + pe_
    qh = jnp.concatenate(
        [q_, jnp.broadcast_to(coords[None], (H, N, CDIM))], axis=-1)
    proj = jnp.einsum('hnd,rhd->rhn', qh, alpha) + beta[..., None]
    idx = jnp.argsort(proj, axis=-1).astype(jnp.int32)

    idx_g = (idx + (jnp.arange(H, dtype=jnp.int32) * N)[None, :, None])
    idx_g = idx_g.reshape(RH * N)
    idx_s = (idx + (jnp.arange(RH, dtype=jnp.int32) * N).reshape(R, H, 1))
    idx_s = idx_s.reshape(RH * N)

    qks, vvs = _sc_gather2(qk.reshape(H * N, 2 * D), vv.reshape(H * N, 2 * D),
                           idx_g)
    nblocks = RH * (N // BLK)
    qks = qks.reshape(nblocks, BLK, 2 * D)
    vvs = vvs.reshape(nblocks, BLK, 2 * D)
    ob = pl.pallas_call(
        _attn_body,
        grid=(nblocks // BB,),
        in_specs=[
            pl.BlockSpec((BB, BLK, 2 * D), lambda i: (i, 0, 0)),
            pl.BlockSpec((BB, BLK, 2 * D), lambda i: (i, 0, 0)),
        ],
        out_specs=pl.BlockSpec((BB, BLK, 2 * D), lambda i: (i, 0, 0)),
        out_shape=jax.ShapeDtypeStruct((nblocks, BLK, 2 * D), jnp.float32),
    )(qks, vvs)

    o_all = _sc_scatter(ob.reshape(RH * N, 2 * D), idx_s)
    obi = o_all.reshape(R, H, N, 2 * D)

    out = pl.pallas_call(
        _epi_body,
        grid=(grid1,),
        in_specs=[
            pl.BlockSpec((TN, D), lambda i: (i, 0)),
            pl.BlockSpec((H, TN, 2 * D), lambda i: (0, i, 0)),
            pl.BlockSpec((H, TN, 2 * D), lambda i: (0, i, 0)),
            pl.BlockSpec((H, D, D), lambda i: (0, 0, 0)),
            pl.BlockSpec((1, D), lambda i: (0, 0)),
            pl.BlockSpec((1, D), lambda i: (0, 0)),
            pl.BlockSpec((1, D), lambda i: (0, 0)),
            pl.BlockSpec((D, D), lambda i: (0, 0)),
            pl.BlockSpec((1, D), lambda i: (0, 0)),
            pl.BlockSpec((D, D), lambda i: (0, 0)),
            pl.BlockSpec((1, D), lambda i: (0, 0)),
        ],
        out_specs=pl.BlockSpec((TN, D), lambda i: (i, 0)),
        out_shape=jax.ShapeDtypeStruct((N, D), jnp.float32),
    )(x, obi[0], obi[1], wot, w_o_b.reshape(1, D),
      ln2_s.reshape(1, D), ln2_b.reshape(1, D),
      ff_w1.T, ff_b1.reshape(1, D), ff_w2.T, ff_b2.reshape(1, D))
    return out


# speedup vs baseline: 12.0332x; 1.3549x over previous
import functools

import jax
import jax.numpy as jnp
from jax import lax
from jax.experimental import pallas as pl
from jax.experimental.pallas import tpu as pltpu
from jax.experimental.pallas import tpu_sc as plsc

N = 16384
H = 8
D = 64
R = 2
BLK = 128
CDIM = 6
NW = 8
RH = R * H

TN = 1024
BB = 16

NWORK = 32
ROWS_PER_W = RH * N // NWORK
GW = 512
NWIN = ROWS_PER_W // GW

_SC_MESH = plsc.VectorSubcoreMesh(core_axis_name="c", subcore_axis_name="s")


@functools.partial(
    pl.kernel,
    mesh=_SC_MESH,
    out_type=[jax.ShapeDtypeStruct((RH * N, 2 * D), jnp.float32)] * 2,
    scratch_types=[
        pltpu.VMEM((GW,), jnp.int32),
        pltpu.VMEM((GW, 2 * D), jnp.float32),
        pltpu.SemaphoreType.DMA,
    ],
)
def _sc_gather2(qk_hbm, vv_hbm, idx_hbm, qks_hbm, vvs_hbm, idxv, buf, sem):
    wid = lax.axis_index("s") * 2 + lax.axis_index("c")
    base = wid * ROWS_PER_W
    for w in range(NWIN):
        row = base + w * GW
        pltpu.sync_copy(idx_hbm.at[pl.ds(row, GW)], idxv)
        for tab, dst in ((qk_hbm, qks_hbm), (vv_hbm, vvs_hbm)):
            pltpu.async_copy(tab.at[idxv], buf, sem).wait()
            pltpu.sync_copy(buf, dst.at[pl.ds(row, GW)])


@functools.partial(
    pl.kernel,
    mesh=_SC_MESH,
    out_type=jax.ShapeDtypeStruct((RH * N, 2 * D), jnp.float32),
    scratch_types=[
        pltpu.VMEM((GW,), jnp.int32),
        pltpu.VMEM((GW, 2 * D), jnp.float32),
        pltpu.SemaphoreType.DMA,
    ],
)
def _sc_scatter(ob_hbm, idx_hbm, out_hbm, idxv, buf, sem):
    wid = lax.axis_index("s") * 2 + lax.axis_index("c")
    base = wid * ROWS_PER_W
    for w in range(NWIN):
        row = base + w * GW
        pltpu.sync_copy(idx_hbm.at[pl.ds(row, GW)], idxv)
        pltpu.sync_copy(ob_hbm.at[pl.ds(row, GW)], buf)
        pltpu.async_copy(buf, out_hbm.at[idxv], sem).wait()


def _prep_body(x_ref, c_ref, wqt_ref, wkt_ref, wvt_ref, wpe_ref, bpe_ref,
               s_ref, b_ref, qk_ref, vv_ref):
    x = x_ref[...]
    c = c_ref[...]
    m = jnp.mean(x, axis=-1, keepdims=True)
    v_ = jnp.mean((x - m) * (x - m), axis=-1, keepdims=True)
    xn = (x - m) / jnp.sqrt(v_ + 1e-5) * s_ref[...] + b_ref[...]

    pe = jnp.dot(c[:, :CDIM - 1], wpe_ref[...],
                 preferred_element_type=jnp.float32) + bpe_ref[...]
    qf = jnp.dot(xn, wqt_ref[...], preferred_element_type=jnp.float32) + pe
    kf = jnp.dot(xn, wkt_ref[...], preferred_element_type=jnp.float32) + pe
    vf = jnp.dot(xn, wvt_ref[...], preferred_element_type=jnp.float32)

    for h in range(H):
        qk_ref[h, :, :D] = qf[:, h * D:(h + 1) * D]
        qk_ref[h, :, D:] = kf[:, h * D:(h + 1) * D]
        vv_ref[h, :, :D] = vf[:, h * D:(h + 1) * D]
        vv_ref[h, :, D:] = vf[:, h * D:(h + 1) * D]


def _attn_body(qk_ref, v_ref, o_ref):
    q = qk_ref[:, :, :D]
    k = qk_ref[:, :, D:]
    s = lax.dot_general(q, k, (((2,), (2,)), ((0,), (0,))),
                        preferred_element_type=jnp.float32) * (1.0 / 8.0)
    mx = jnp.max(s, axis=-1, keepdims=True)
    p = jnp.exp(s - mx)
    l = jnp.sum(p, axis=-1, keepdims=True)
    o = lax.dot_general(p / l, v_ref[:, :, :D], (((2,), (1,)), ((0,), (0,))),
                        preferred_element_type=jnp.float32)
    o_ref[:, :, :D] = o
    o_ref[:, :, D:] = o


def _epi_body(x_ref, o0_ref, o1_ref, wot_ref, wob_ref, s2_ref, b2_ref,
              w1t_ref, b1_ref, w2t_ref, b2f_ref, out_ref):
    acc = jnp.zeros((TN, D), jnp.float32)
    for h in range(H):
        oh = (o0_ref[h, :, :D] + o1_ref[h, :, :D]) * 0.5
        acc = acc + jnp.dot(oh, wot_ref[h], preferred_element_type=jnp.float32)
    x1 = x_ref[...] + acc + wob_ref[...]
    m = jnp.mean(x1, axis=-1, keepdims=True)
    v_ = jnp.mean((x1 - m) * (x1 - m), axis=-1, keepdims=True)
    x2 = (x1 - m) / jnp.sqrt(v_ + 1e-5) * s2_ref[...] + b2_ref[...]
    h1 = jnp.maximum(jnp.dot(x2, w1t_ref[...],
                             preferred_element_type=jnp.float32) + b1_ref[...], 0.0)
    ff = jnp.dot(h1, w2t_ref[...], preferred_element_type=jnp.float32) + b2f_ref[...]
    out_ref[...] = x1 + ff


def kernel(x, coords, w_q, w_k, w_v, w_rpe_w, w_rpe_b, w_o_w, w_o_b,
           ln1_s, ln1_b, ln2_s, ln2_b, ff_w1, ff_b1, ff_w2, ff_b2, alpha, beta):
    wqt = w_q.T
    wkt = w_k.T
    wvt = w_v.T
    wpe = w_rpe_w.reshape(H * D, CDIM - 1, NW).sum(-1).T
    bpe = w_rpe_b.reshape(1, H * D)
    wot = w_o_w.T.reshape(H, D, D)

    grid1 = N // TN
    qk, vv = pl.pallas_call(
        _prep_body,
        grid=(grid1,),
        in_specs=[
            pl.BlockSpec((TN, D), lambda i: (i, 0)),
            pl.BlockSpec((TN, CDIM), lambda i: (i, 0)),
            pl.BlockSpec((D, H * D), lambda i: (0, 0)),
            pl.BlockSpec((D, H * D), lambda i: (0, 0)),
            pl.BlockSpec((D, H * D), lambda i: (0, 0)),
            pl.BlockSpec((CDIM - 1, H * D), lambda i: (0, 0)),
            pl.BlockSpec((1, H * D), lambda i: (0, 0)),
            pl.BlockSpec((1, D), lambda i: (0, 0)),
            pl.BlockSpec((1, D), lambda i: (0, 0)),
        ],
        out_specs=[
            pl.BlockSpec((H, TN, 2 * D), lambda i: (0, i, 0)),
            pl.BlockSpec((H, TN, 2 * D), lambda i: (0, i, 0)),
        ],
        out_shape=[
            jax.ShapeDtypeStruct((H, N, 2 * D), jnp.float32),
            jax.ShapeDtypeStruct((H, N, 2 * D), jnp.float32),
        ],
    )(x, coords, wqt, wkt, wvt, wpe, bpe,
      ln1_s.reshape(1, D), ln1_b.reshape(1, D))

    m_ = jnp.mean(x, axis=-1, keepdims=True)
    v_ = jnp.var(x, axis=-1, keepdims=True)
    xn_ = (x - m_) / jnp.sqrt(v_ + 1e-5) * ln1_s + ln1_b
    q_ = (xn_ @ w_q.T).reshape(N, H, D).transpose(1, 0, 2)
    dist_feat = jnp.repeat(coords[:, : CDIM - 1], NW, axis=1)
    pe_ = (dist_feat @ w_rpe_w.T + w_rpe_b).reshape(N, H, D).transpose(1, 0, 2)
    q_ = q_ + pe_
    qh = jnp.concatenate(
        [q_, jnp.broadcast_to(coords[None], (H, N, CDIM))], axis=-1)
    proj = jnp.einsum('hnd,rhd->rhn', qh, alpha) + beta[..., None]
    idx = jnp.argsort(proj, axis=-1).astype(jnp.int32)

    idx_g = (idx + (jnp.arange(H, dtype=jnp.int32) * N)[None, :, None])
    idx_g = idx_g.reshape(RH * N)
    idx_s = (idx + (jnp.arange(RH, dtype=jnp.int32) * N).reshape(R, H, 1))
    idx_s = idx_s.reshape(RH * N)

    qks, vvs = _sc_gather2(qk.reshape(H * N, 2 * D), vv.reshape(H * N, 2 * D),
                           idx_g)
    nblocks = RH * (N // BLK)
    qks = qks.reshape(nblocks, BLK, 2 * D)
    vvs = vvs.reshape(nblocks, BLK, 2 * D)
    ob = pl.pallas_call(
        _attn_body,
        grid=(nblocks // BB,),
        in_specs=[
            pl.BlockSpec((BB, BLK, 2 * D), lambda i: (i, 0, 0)),
            pl.BlockSpec((BB, BLK, 2 * D), lambda i: (i, 0, 0)),
        ],
        out_specs=pl.BlockSpec((BB, BLK, 2 * D), lambda i: (i, 0, 0)),
        out_shape=jax.ShapeDtypeStruct((nblocks, BLK, 2 * D), jnp.float32),
    )(qks, vvs)

    o_all = _sc_scatter(ob.reshape(RH * N, 2 * D), idx_s)
    obi = o_all.reshape(R, H, N, 2 * D)

    out = pl.pallas_call(
        _epi_body,
        grid=(grid1,),
        in_specs=[
            pl.BlockSpec((TN, D), lambda i: (i, 0)),
            pl.BlockSpec((H, TN, 2 * D), lambda i: (0, i, 0)),
            pl.BlockSpec((H, TN, 2 * D), lambda i: (0, i, 0)),
            pl.BlockSpec((H, D, D), lambda i: (0, 0, 0)),
            pl.BlockSpec((1, D), lambda i: (0, 0)),
            pl.BlockSpec((1, D), lambda i: (0, 0)),
            pl.BlockSpec((1, D), lambda i: (0, 0)),
            pl.BlockSpec((D, D), lambda i: (0, 0)),
            pl.BlockSpec((1, D), lambda i: (0, 0)),
            pl.BlockSpec((D, D), lambda i: (0, 0)),
            pl.BlockSpec((1, D), lambda i: (0, 0)),
        ],
        out_specs=pl.BlockSpec((TN, D), lambda i: (i, 0)),
        out_shape=jax.ShapeDtypeStruct((N, D), jnp.float32),
    )(x, obi[0], obi[1], wot, w_o_b.reshape(1, D),
      ln2_s.reshape(1, D), ln2_b.reshape(1, D),
      ff_w1.T, ff_b1.reshape(1, D), ff_w2.T, ff_b2.reshape(1, D))
    return out


# speedup vs baseline: 13.5607x; 1.1269x over previous
import functools

import jax
import jax.numpy as jnp
from jax import lax
from jax.experimental import pallas as pl
from jax.experimental.pallas import tpu as pltpu
from jax.experimental.pallas import tpu_sc as plsc

N = 16384
H = 8
D = 64
R = 2
BLK = 128
CDIM = 6
NW = 8
RH = R * H

TN = 1024
BB = 16

NWORK = 32
GW = 512
ROWS_HALF = H * N // NWORK
NWIN_HALF = ROWS_HALF // GW

_SC_MESH = plsc.VectorSubcoreMesh(core_axis_name="c", subcore_axis_name="s")


@functools.partial(
    pl.kernel,
    mesh=_SC_MESH,
    out_type=[jax.ShapeDtypeStruct((H * N, 2 * D), jnp.float32)] * 2,
    scratch_types=[
        pltpu.VMEM((GW,), jnp.int32),
        pltpu.VMEM((GW, 2 * D), jnp.float32),
        pltpu.SemaphoreType.DMA,
    ],
)
def _sc_gather2(qk_hbm, vv_hbm, idx_hbm, qks_hbm, vvs_hbm, idxv, buf, sem):
    wid = lax.axis_index("s") * 2 + lax.axis_index("c")
    base = wid * ROWS_HALF
    for w in range(NWIN_HALF):
        row = base + w * GW
        pltpu.sync_copy(idx_hbm.at[pl.ds(row, GW)], idxv)
        for tab, dst in ((qk_hbm, qks_hbm), (vv_hbm, vvs_hbm)):
            pltpu.async_copy(tab.at[idxv], buf, sem).wait()
            pltpu.sync_copy(buf, dst.at[pl.ds(row, GW)])


@functools.partial(
    pl.kernel,
    mesh=_SC_MESH,
    out_type=jax.ShapeDtypeStruct((H * N, 2 * D), jnp.float32),
    scratch_types=[
        pltpu.VMEM((GW,), jnp.int32),
        pltpu.VMEM((GW, 2 * D), jnp.float32),
        pltpu.SemaphoreType.DMA,
    ],
)
def _sc_scatter(ob_hbm, idx_hbm, out_hbm, idxv, buf, sem):
    wid = lax.axis_index("s") * 2 + lax.axis_index("c")
    base = wid * ROWS_HALF
    for w in range(NWIN_HALF):
        row = base + w * GW
        pltpu.sync_copy(idx_hbm.at[pl.ds(row, GW)], idxv)
        pltpu.sync_copy(ob_hbm.at[pl.ds(row, GW)], buf)
        pltpu.async_copy(buf, out_hbm.at[idxv], sem).wait()


def _prep_body(x_ref, c_ref, wqt_ref, wkt_ref, wvt_ref, wpe_ref, bpe_ref,
               s_ref, b_ref, qk_ref, vv_ref):
    x = x_ref[...]
    c = c_ref[...]
    m = jnp.mean(x, axis=-1, keepdims=True)
    v_ = jnp.mean((x - m) * (x - m), axis=-1, keepdims=True)
    xn = (x - m) / jnp.sqrt(v_ + 1e-5) * s_ref[...] + b_ref[...]

    pe = jnp.dot(c[:, :CDIM - 1], wpe_ref[...],
                 preferred_element_type=jnp.float32) + bpe_ref[...]
    qf = jnp.dot(xn, wqt_ref[...], preferred_element_type=jnp.float32) + pe
    kf = jnp.dot(xn, wkt_ref[...], preferred_element_type=jnp.float32) + pe
    vf = jnp.dot(xn, wvt_ref[...], preferred_element_type=jnp.float32)

    for h in range(H):
        qk_ref[h, :, :D] = qf[:, h * D:(h + 1) * D]
        qk_ref[h, :, D:] = kf[:, h * D:(h + 1) * D]
        vv_ref[h, :, :D] = vf[:, h * D:(h + 1) * D]
        vv_ref[h, :, D:] = vf[:, h * D:(h + 1) * D]


def _attn_body(qk_ref, v_ref, o_ref):
    q = qk_ref[:, :, :D]
    k = qk_ref[:, :, D:]
    s = lax.dot_general(q, k, (((2,), (2,)), ((0,), (0,))),
                        preferred_element_type=jnp.float32) * (1.0 / 8.0)
    mx = jnp.max(s, axis=-1, keepdims=True)
    p = jnp.exp(s - mx)
    l = jnp.sum(p, axis=-1, keepdims=True)
    o = lax.dot_general(p / l, v_ref[:, :, :D], (((2,), (1,)), ((0,), (0,))),
                        preferred_element_type=jnp.float32)
    o_ref[:, :, :D] = o
    o_ref[:, :, D:] = o


def _epi_body(x_ref, o0_ref, o1_ref, wot_ref, wob_ref, s2_ref, b2_ref,
              w1t_ref, b1_ref, w2t_ref, b2f_ref, out_ref):
    acc = jnp.zeros((TN, D), jnp.float32)
    for h in range(H):
        oh = (o0_ref[h, :, :D] + o1_ref[h, :, :D]) * 0.5
        acc = acc + jnp.dot(oh, wot_ref[h], preferred_element_type=jnp.float32)
    x1 = x_ref[...] + acc + wob_ref[...]
    m = jnp.mean(x1, axis=-1, keepdims=True)
    v_ = jnp.mean((x1 - m) * (x1 - m), axis=-1, keepdims=True)
    x2 = (x1 - m) / jnp.sqrt(v_ + 1e-5) * s2_ref[...] + b2_ref[...]
    h1 = jnp.maximum(jnp.dot(x2, w1t_ref[...],
                             preferred_element_type=jnp.float32) + b1_ref[...], 0.0)
    ff = jnp.dot(h1, w2t_ref[...], preferred_element_type=jnp.float32) + b2f_ref[...]
    out_ref[...] = x1 + ff


def kernel(x, coords, w_q, w_k, w_v, w_rpe_w, w_rpe_b, w_o_w, w_o_b,
           ln1_s, ln1_b, ln2_s, ln2_b, ff_w1, ff_b1, ff_w2, ff_b2, alpha, beta):
    wqt = w_q.T
    wkt = w_k.T
    wvt = w_v.T
    wpe = w_rpe_w.reshape(H * D, CDIM - 1, NW).sum(-1).T
    bpe = w_rpe_b.reshape(1, H * D)
    wot = w_o_w.T.reshape(H, D, D)

    grid1 = N // TN
    qk, vv = pl.pallas_call(
        _prep_body,
        grid=(grid1,),
        in_specs=[
            pl.BlockSpec((TN, D), lambda i: (i, 0)),
            pl.BlockSpec((TN, CDIM), lambda i: (i, 0)),
            pl.BlockSpec((D, H * D), lambda i: (0, 0)),
            pl.BlockSpec((D, H * D), lambda i: (0, 0)),
            pl.BlockSpec((D, H * D), lambda i: (0, 0)),
            pl.BlockSpec((CDIM - 1, H * D), lambda i: (0, 0)),
            pl.BlockSpec((1, H * D), lambda i: (0, 0)),
            pl.BlockSpec((1, D), lambda i: (0, 0)),
            pl.BlockSpec((1, D), lambda i: (0, 0)),
        ],
        out_specs=[
            pl.BlockSpec((H, TN, 2 * D), lambda i: (0, i, 0)),
            pl.BlockSpec((H, TN, 2 * D), lambda i: (0, i, 0)),
        ],
        out_shape=[
            jax.ShapeDtypeStruct((H, N, 2 * D), jnp.float32),
            jax.ShapeDtypeStruct((H, N, 2 * D), jnp.float32),
        ],
    )(x, coords, wqt, wkt, wvt, wpe, bpe,
      ln1_s.reshape(1, D), ln1_b.reshape(1, D))

    m_ = jnp.mean(x, axis=-1, keepdims=True)
    v_ = jnp.var(x, axis=-1, keepdims=True)
    xn_ = (x - m_) / jnp.sqrt(v_ + 1e-5) * ln1_s + ln1_b
    q_ = (xn_ @ w_q.T).reshape(N, H, D).transpose(1, 0, 2)
    dist_feat = jnp.repeat(coords[:, : CDIM - 1], NW, axis=1)
    pe_ = (dist_feat @ w_rpe_w.T + w_rpe_b).reshape(N, H, D).transpose(1, 0, 2)
    q_ = q_ + pe_
    qh = jnp.concatenate(
        [q_, jnp.broadcast_to(coords[None], (H, N, CDIM))], axis=-1)
    proj = jnp.einsum('hnd,rhd->rhn', qh, alpha) + beta[..., None]
    idx = jnp.argsort(proj, axis=-1).astype(jnp.int32)

    idx_g = (idx + (jnp.arange(H, dtype=jnp.int32) * N)[None, :, None])
    idx_g = idx_g.reshape(R, H * N)

    qk2 = qk.reshape(H * N, 2 * D)
    vv2 = vv.reshape(H * N, 2 * D)
    nb_half = H * (N // BLK)

    def _attn(qks, vvs):
        return pl.pallas_call(
            _attn_body,
            grid=(nb_half // BB,),
            in_specs=[
                pl.BlockSpec((BB, BLK, 2 * D), lambda i: (i, 0, 0)),
                pl.BlockSpec((BB, BLK, 2 * D), lambda i: (i, 0, 0)),
            ],
            out_specs=pl.BlockSpec((BB, BLK, 2 * D), lambda i: (i, 0, 0)),
            out_shape=jax.ShapeDtypeStruct((nb_half, BLK, 2 * D), jnp.float32),
        )(qks.reshape(nb_half, BLK, 2 * D), vvs.reshape(nb_half, BLK, 2 * D))

    o_r = []
    for r in range(R):
        qks, vvs = _sc_gather2(qk2, vv2, idx_g[r])
        ob = _attn(qks, vvs)
        o_r.append(_sc_scatter(ob.reshape(H * N, 2 * D), idx_g[r]))
    obi0 = o_r[0].reshape(H, N, 2 * D)
    obi1 = o_r[1].reshape(H, N, 2 * D)

    out = pl.pallas_call(
        _epi_body,
        grid=(grid1,),
        in_specs=[
            pl.BlockSpec((TN, D), lambda i: (i, 0)),
            pl.BlockSpec((H, TN, 2 * D), lambda i: (0, i, 0)),
            pl.BlockSpec((H, TN, 2 * D), lambda i: (0, i, 0)),
            pl.BlockSpec((H, D, D), lambda i: (0, 0, 0)),
            pl.BlockSpec((1, D), lambda i: (0, 0)),
            pl.BlockSpec((1, D), lambda i: (0, 0)),
            pl.BlockSpec((1, D), lambda i: (0, 0)),
            pl.BlockSpec((D, D), lambda i: (0, 0)),
            pl.BlockSpec((1, D), lambda i: (0, 0)),
            pl.BlockSpec((D, D), lambda i: (0, 0)),
            pl.BlockSpec((1, D), lambda i: (0, 0)),
        ],
        out_specs=pl.BlockSpec((TN, D), lambda i: (i, 0)),
        out_shape=jax.ShapeDtypeStruct((N, D), jnp.float32),
    )(x, obi0, obi1, wot, w_o_b.reshape(1, D),
      ln2_s.reshape(1, D), ln2_b.reshape(1, D),
      ff_w1.T, ff_b1.reshape(1, D), ff_w2.T, ff_b2.reshape(1, D))
    return out


# speedup vs baseline: 21.5204x; 1.5870x over previous
import functools

import jax
import jax.numpy as jnp
from jax import lax
from jax.experimental import pallas as pl
from jax.experimental.pallas import tpu as pltpu
from jax.experimental.pallas import tpu_sc as plsc

N = 16384
H = 8
D = 64
R = 2
BLK = 128
CDIM = 6
NW = 8
RH = R * H

TN = 1024
BB = 16

NWORK = 32
GW = 256
ROWS_HALF = H * N // NWORK
NWIN_HALF = ROWS_HALF // GW

_SC_MESH = plsc.VectorSubcoreMesh(core_axis_name="c", subcore_axis_name="s")


@functools.partial(
    pl.kernel,
    mesh=_SC_MESH,
    out_type=[jax.ShapeDtypeStruct((H * N, 2 * D), jnp.float32)] * 2,
    scratch_types=[
        pltpu.VMEM((GW,), jnp.int32),
        pltpu.VMEM((GW, 2 * D), jnp.float32),
        pltpu.VMEM((GW, 2 * D), jnp.float32),
        pltpu.SemaphoreType.DMA,
        pltpu.SemaphoreType.DMA,
        pltpu.SemaphoreType.DMA,
    ],
)
def _sc_gather2(qk_hbm, vv_hbm, idx_hbm, qks_hbm, vvs_hbm,
                idxv, bufq, bufv, gsem, wsemq, wsemv):
    wid = lax.axis_index("s") * 2 + lax.axis_index("c")
    base = wid * ROWS_HALF
    wq = wv = None
    for w in range(NWIN_HALF):
        row = base + w * GW
        pltpu.sync_copy(idx_hbm.at[pl.ds(row, GW)], idxv)
        if wq is not None:
            wq.wait()
        pltpu.async_copy(qk_hbm.at[idxv], bufq, gsem).wait()
        wq = pltpu.async_copy(bufq, qks_hbm.at[pl.ds(row, GW)], wsemq)
        if wv is not None:
            wv.wait()
        pltpu.async_copy(vv_hbm.at[idxv], bufv, gsem).wait()
        wv = pltpu.async_copy(bufv, vvs_hbm.at[pl.ds(row, GW)], wsemv)
    wq.wait()
    wv.wait()


@functools.partial(
    pl.kernel,
    mesh=_SC_MESH,
    out_type=jax.ShapeDtypeStruct((H * N, 2 * D), jnp.float32),
    scratch_types=[
        pltpu.VMEM((GW,), jnp.int32),
        pltpu.VMEM((GW,), jnp.int32),
        pltpu.VMEM((GW, 2 * D), jnp.float32),
        pltpu.VMEM((GW, 2 * D), jnp.float32),
        pltpu.SemaphoreType.DMA,
        pltpu.SemaphoreType.DMA,
    ],
)
def _sc_scatter(ob_hbm, idx_hbm, out_hbm, idxv0, idxv1, buf0, buf1, sem0, sem1):
    wid = lax.axis_index("s") * 2 + lax.axis_index("c")
    base = wid * ROWS_HALF
    idxv = (idxv0, idxv1)
    buf = (buf0, buf1)
    sem = (sem0, sem1)
    pend = [None, None]
    for w in range(NWIN_HALF):
        row = base + w * GW
        p = w % 2
        if pend[p] is not None:
            pend[p].wait()
        pltpu.sync_copy(idx_hbm.at[pl.ds(row, GW)], idxv[p])
        pltpu.sync_copy(ob_hbm.at[pl.ds(row, GW)], buf[p])
        pend[p] = pltpu.async_copy(buf[p], out_hbm.at[idxv[p]], sem[p])
    pend[0].wait()
    pend[1].wait()


def _prep_body(x_ref, c_ref, wqt_ref, wkt_ref, wvt_ref, wpe_ref, bpe_ref,
               s_ref, b_ref, qk_ref, vv_ref):
    x = x_ref[...]
    c = c_ref[...]
    m = jnp.mean(x, axis=-1, keepdims=True)
    v_ = jnp.mean((x - m) * (x - m), axis=-1, keepdims=True)
    xn = (x - m) / jnp.sqrt(v_ + 1e-5) * s_ref[...] + b_ref[...]

    pe = jnp.dot(c[:, :CDIM - 1], wpe_ref[...],
                 preferred_element_type=jnp.float32) + bpe_ref[...]
    qf = jnp.dot(xn, wqt_ref[...], preferred_element_type=jnp.float32) + pe
    kf = jnp.dot(xn, wkt_ref[...], preferred_element_type=jnp.float32) + pe
    vf = jnp.dot(xn, wvt_ref[...], preferred_element_type=jnp.float32)

    for h in range(H):
        qk_ref[h, :, :D] = qf[:, h * D:(h + 1) * D]
        qk_ref[h, :, D:] = kf[:, h * D:(h + 1) * D]
        vv_ref[h, :, :D] = vf[:, h * D:(h + 1) * D]
        vv_ref[h, :, D:] = vf[:, h * D:(h + 1) * D]


def _attn_body(qk_ref, v_ref, o_ref):
    q = qk_ref[:, :, :D]
    k = qk_ref[:, :, D:]
    s = lax.dot_general(q, k, (((2,), (2,)), ((0,), (0,))),
                        preferred_element_type=jnp.float32) * (1.0 / 8.0)
    p = jnp.exp(s)
    l = jnp.sum(p, axis=-1, keepdims=True)
    o = lax.dot_general(p / l, v_ref[:, :, :D], (((2,), (1,)), ((0,), (0,))),
                        preferred_element_type=jnp.float32)
    o_ref[:, :, :D] = o
    o_ref[:, :, D:] = o


def _epi_body(x_ref, o0_ref, o1_ref, wot_ref, wob_ref, s2_ref, b2_ref,
              w1t_ref, b1_ref, w2t_ref, b2f_ref, out_ref):
    acc = jnp.zeros((TN, D), jnp.float32)
    for h in range(H):
        oh = (o0_ref[h, :, :D] + o1_ref[h, :, :D]) * 0.5
        acc = acc + jnp.dot(oh, wot_ref[h], preferred_element_type=jnp.float32)
    x1 = x_ref[...] + acc + wob_ref[...]
    m = jnp.mean(x1, axis=-1, keepdims=True)
    v_ = jnp.mean((x1 - m) * (x1 - m), axis=-1, keepdims=True)
    x2 = (x1 - m) / jnp.sqrt(v_ + 1e-5) * s2_ref[...] + b2_ref[...]
    h1 = jnp.maximum(jnp.dot(x2, w1t_ref[...],
                             preferred_element_type=jnp.float32) + b1_ref[...], 0.0)
    ff = jnp.dot(h1, w2t_ref[...], preferred_element_type=jnp.float32) + b2f_ref[...]
    out_ref[...] = x1 + ff


def kernel(x, coords, w_q, w_k, w_v, w_rpe_w, w_rpe_b, w_o_w, w_o_b,
           ln1_s, ln1_b, ln2_s, ln2_b, ff_w1, ff_b1, ff_w2, ff_b2, alpha, beta):
    wqt = w_q.T
    wkt = w_k.T
    wvt = w_v.T
    wpe = w_rpe_w.reshape(H * D, CDIM - 1, NW).sum(-1).T
    bpe = w_rpe_b.reshape(1, H * D)
    wot = w_o_w.T.reshape(H, D, D)

    grid1 = N // TN
    qk, vv = pl.pallas_call(
        _prep_body,
        grid=(grid1,),
        in_specs=[
            pl.BlockSpec((TN, D), lambda i: (i, 0)),
            pl.BlockSpec((TN, CDIM), lambda i: (i, 0)),
            pl.BlockSpec((D, H * D), lambda i: (0, 0)),
            pl.BlockSpec((D, H * D), lambda i: (0, 0)),
            pl.BlockSpec((D, H * D), lambda i: (0, 0)),
            pl.BlockSpec((CDIM - 1, H * D), lambda i: (0, 0)),
            pl.BlockSpec((1, H * D), lambda i: (0, 0)),
            pl.BlockSpec((1, D), lambda i: (0, 0)),
            pl.BlockSpec((1, D), lambda i: (0, 0)),
        ],
        out_specs=[
            pl.BlockSpec((H, TN, 2 * D), lambda i: (0, i, 0)),
            pl.BlockSpec((H, TN, 2 * D), lambda i: (0, i, 0)),
        ],
        out_shape=[
            jax.ShapeDtypeStruct((H, N, 2 * D), jnp.float32),
            jax.ShapeDtypeStruct((H, N, 2 * D), jnp.float32),
        ],
    )(x, coords, wqt, wkt, wvt, wpe, bpe,
      ln1_s.reshape(1, D), ln1_b.reshape(1, D))

    m_ = jnp.mean(x, axis=-1, keepdims=True)
    v_ = jnp.var(x, axis=-1, keepdims=True)
    xn_ = (x - m_) / jnp.sqrt(v_ + 1e-5) * ln1_s + ln1_b
    q_ = (xn_ @ w_q.T).reshape(N, H, D).transpose(1, 0, 2)
    dist_feat = jnp.repeat(coords[:, : CDIM - 1], NW, axis=1)
    pe_ = (dist_feat @ w_rpe_w.T + w_rpe_b).reshape(N, H, D).transpose(1, 0, 2)
    q_ = q_ + pe_
    qh = jnp.concatenate(
        [q_, jnp.broadcast_to(coords[None], (H, N, CDIM))], axis=-1)
    proj = jnp.einsum('hnd,rhd->rhn', qh, alpha) + beta[..., None]

    qk2 = qk.reshape(H * N, 2 * D)
    vv2 = vv.reshape(H * N, 2 * D)
    nb_half = H * (N // BLK)

    def _attn(qks, vvs):
        return pl.pallas_call(
            _attn_body,
            grid=(nb_half // BB,),
            in_specs=[
                pl.BlockSpec((BB, BLK, 2 * D), lambda i: (i, 0, 0)),
                pl.BlockSpec((BB, BLK, 2 * D), lambda i: (i, 0, 0)),
            ],
            out_specs=pl.BlockSpec((BB, BLK, 2 * D), lambda i: (i, 0, 0)),
            out_shape=jax.ShapeDtypeStruct((nb_half, BLK, 2 * D), jnp.float32),
        )(qks.reshape(nb_half, BLK, 2 * D), vvs.reshape(nb_half, BLK, 2 * D))

    head_off = (jnp.arange(H, dtype=jnp.int32) * N)[:, None]
    o_r = []
    for r in range(R):
        idx_r = jnp.argsort(proj[r], axis=-1).astype(jnp.int32)
        idx_g_r = (idx_r + head_off).reshape(H * N)
        qks, vvs = _sc_gather2(qk2, vv2, idx_g_r)
        ob = _attn(qks, vvs)
        o_r.append(_sc_scatter(ob.reshape(H * N, 2 * D), idx_g_r))
    obi0 = o_r[0].reshape(H, N, 2 * D)
    obi1 = o_r[1].reshape(H, N, 2 * D)

    out = pl.pallas_call(
        _epi_body,
        grid=(grid1,),
        in_specs=[
            pl.BlockSpec((TN, D), lambda i: (i, 0)),
            pl.BlockSpec((H, TN, 2 * D), lambda i: (0, i, 0)),
            pl.BlockSpec((H, TN, 2 * D), lambda i: (0, i, 0)),
            pl.BlockSpec((H, D, D), lambda i: (0, 0, 0)),
            pl.BlockSpec((1, D), lambda i: (0, 0)),
            pl.BlockSpec((1, D), lambda i: (0, 0)),
            pl.BlockSpec((1, D), lambda i: (0, 0)),
            pl.BlockSpec((D, D), lambda i: (0, 0)),
            pl.BlockSpec((1, D), lambda i: (0, 0)),
            pl.BlockSpec((D, D), lambda i: (0, 0)),
            pl.BlockSpec((1, D), lambda i: (0, 0)),
        ],
        out_specs=pl.BlockSpec((TN, D), lambda i: (i, 0)),
        out_shape=jax.ShapeDtypeStruct((N, D), jnp.float32),
    )(x, obi0, obi1, wot, w_o_b.reshape(1, D),
      ln2_s.reshape(1, D), ln2_b.reshape(1, D),
      ff_w1.T, ff_b1.reshape(1, D), ff_w2.T, ff_b2.reshape(1, D))
    return out
